# Initial kernel scaffold; baseline (speedup 1.0000x reference)
import jax
import jax.numpy as jnp
from jax.experimental import pallas as pl


def kernel(x, edge_index, edge_label_index, W1, b1, W2, b2):
    raise NotImplementedError("write your pallas kernel here")


# speedup vs baseline: 8.1087x; 8.1087x over previous
import functools

import jax
import jax.numpy as jnp
from jax import lax
from jax.experimental import pallas as pl
from jax.experimental.pallas import tpu as pltpu
from jax.experimental.pallas import tpu_sc as plsc

N = 10000
E = 320000
D = 128
H = 128
P = 100000

NP = 10240
NC = 2
NS = 16
NW = NC * NS
CH = 128
EC = 80
E_PAD = NW * EC * CH
PC = 26
P_PAD = NW * PC * CH
RPT = NP // NS

BR = 1024

_mesh = plsc.VectorSubcoreMesh(core_axis_name="c", subcore_axis_name="s")
_sc_params = pltpu.CompilerParams(needs_layout_passes=False)


@functools.partial(
    pl.kernel,
    out_type=jax.ShapeDtypeStruct((NC, NP, 16), jnp.float32),
    mesh=_mesh,
    compiler_params=_sc_params,
    scratch_types=[
        pltpu.VMEM_SHARED((NP, 16), jnp.float32),
        pltpu.VMEM((2, CH), jnp.int32),
        pltpu.VMEM((CH, 16), jnp.float32),
        pltpu.VMEM((RPT, 16), jnp.float32),
    ],
)
def _sc_degree(dst_hbm, deg_hbm, acc16, idx_v, ones_v, zb):
    cid = lax.axis_index("c")
    sid = lax.axis_index("s")
    wid = sid * NC + cid

    @pl.loop(0, RPT)
    def _(r):
        zb[r] = jnp.zeros((16,), jnp.float32)

    @pl.loop(0, CH)
    def _(r):
        ones_v[r] = jnp.ones((16,), jnp.float32)

    pltpu.sync_copy(zb, acc16.at[pl.ds(sid * RPT, RPT)])
    plsc.subcore_barrier()

    ebase = wid * (EC * CH)

    @pl.loop(0, EC)
    def _(c):
        pltpu.sync_copy(dst_hbm.at[pl.ds(ebase + c * CH, CH)], idx_v.at[0])
        pltpu.sync_copy(ones_v, acc16.at[idx_v.at[0]], add=True)

    plsc.subcore_barrier()
    pltpu.sync_copy(acc16.at[pl.ds(sid * RPT, RPT)],
                    deg_hbm.at[cid, pl.ds(sid * RPT, RPT)])


@functools.partial(
    pl.kernel,
    out_type=jax.ShapeDtypeStruct((NC, NP, H), jnp.float32),
    mesh=_mesh,
    compiler_params=_sc_params,
    scratch_types=[
        pltpu.VMEM_SHARED((NP, H), jnp.float32),
        pltpu.VMEM((2, CH), jnp.int32),
        pltpu.VMEM((2, CH), jnp.int32),
        pltpu.VMEM((CH, H), jnp.float32),
        pltpu.VMEM((CH, H), jnp.float32),
        pltpu.SemaphoreType.DMA,
        pltpu.SemaphoreType.DMA,
        pltpu.SemaphoreType.DMA,
        pltpu.SemaphoreType.DMA,
    ],
)
def _sc_edge_pass(src_hbm, dst_hbm, xws_hbm, parts_hbm,
                  acc, ij_a, ij_b, rows_a, rows_b,
                  gsem_a, gsem_b, isem_a, isem_b):
    cid = lax.axis_index("c")
    sid = lax.axis_index("s")
    wid = sid * NC + cid

    @pl.loop(0, CH)
    def _(r):
        for v in range(H // 16):
            rows_a[r, pl.ds(v * 16, 16)] = jnp.zeros((16,), jnp.float32)

    for k in range(RPT // CH):
        pltpu.sync_copy(rows_a, acc.at[pl.ds(sid * RPT + k * CH, CH)])
    plsc.subcore_barrier()

    rbufs = (rows_a, rows_b)
    gsems = (gsem_a, gsem_b)
    ibufs = (ij_a, ij_b)
    isems = (isem_a, isem_b)
    ebase = wid * EC * CH

    def ij_start(c, buf, sem):
        pltpu.async_copy(src_hbm.at[pl.ds(ebase + c * CH, CH)], buf.at[0], sem)
        pltpu.async_copy(dst_hbm.at[pl.ds(ebase + c * CH, CH)], buf.at[1], sem)

    def ij_wait(c, buf, sem):
        pltpu.make_async_copy(src_hbm.at[pl.ds(ebase + c * CH, CH)],
                              buf.at[0], sem).wait()
        pltpu.make_async_copy(dst_hbm.at[pl.ds(ebase + c * CH, CH)],
                              buf.at[1], sem).wait()

    pltpu.sync_copy(src_hbm.at[pl.ds(ebase, CH)], ij_a.at[0])
    pltpu.sync_copy(dst_hbm.at[pl.ds(ebase, CH)], ij_a.at[1])
    pltpu.sync_copy(src_hbm.at[pl.ds(ebase + CH, CH)], ij_b.at[0])
    pltpu.sync_copy(dst_hbm.at[pl.ds(ebase + CH, CH)], ij_b.at[1])
    pltpu.async_copy(xws_hbm.at[ij_a.at[0]], rows_a, gsem_a)

    @pl.loop(0, EC, step=2)
    def _(c0):
        for b in range(2):
            c = c0 + b

            @pl.when((c > 0) & (c + 1 < EC))
            def _():
                ij_wait(c + 1, ibufs[1 - b], isems[1 - b])

            @pl.when(c + 1 < EC)
            def _():
                pltpu.async_copy(xws_hbm.at[ibufs[1 - b].at[0]],
                                 rbufs[1 - b], gsems[1 - b])

            pltpu.make_async_copy(xws_hbm.at[ibufs[b].at[0]], rbufs[b],
                                  gsems[b]).wait()
            pltpu.sync_copy(rbufs[b], acc.at[ibufs[b].at[1]], add=True)

            @pl.when(c + 2 < EC)
            def _():
                ij_start(c + 2, ibufs[b], isems[b])

    plsc.subcore_barrier()
    pltpu.sync_copy(acc.at[pl.ds(sid * RPT, RPT)],
                    parts_hbm.at[cid, pl.ds(sid * RPT, RPT)])


@functools.partial(
    pl.kernel,
    out_type=jax.ShapeDtypeStruct((P_PAD,), jnp.float32),
    mesh=_mesh,
    compiler_params=_sc_params,
    scratch_types=[
        pltpu.VMEM((2, CH), jnp.int32),
        pltpu.VMEM((2, CH), jnp.int32),
        pltpu.VMEM((CH, H), jnp.float32),
        pltpu.VMEM((CH, H), jnp.float32),
        pltpu.VMEM((CH,), jnp.float32),
        pltpu.SemaphoreType.DMA,
        pltpu.SemaphoreType.DMA,
    ],
)
def _sc_decode(sidx_hbm, didx_hbm, z_hbm, out_hbm,
               si, di, zs, zd, rbuf, sem1, sem2):
    cid = lax.axis_index("c")
    sid = lax.axis_index("s")
    wid = sid * NC + cid
    pbase = wid * (PC * CH)
    rvecs = [lax.iota(jnp.int32, 16) + 16 * g for g in range(CH // 16)]

    @pl.loop(0, PC)
    def _(c):
        off = pbase + c * CH
        pltpu.sync_copy(sidx_hbm.at[pl.ds(off, CH)], si.at[0])
        pltpu.sync_copy(didx_hbm.at[pl.ds(off, CH)], di.at[0])
        a = pltpu.async_copy(z_hbm.at[si.at[0]], zs, sem1)
        b = pltpu.async_copy(z_hbm.at[di.at[0]], zd, sem2)
        a.wait()
        b.wait()
        zero = jnp.zeros((16,), jnp.float32)

        @pl.loop(0, H, init_carry=tuple(zero for _ in range(CH // 16)),
                 unroll=4)
        def accs(j, carry):
            cols = jnp.full((16,), j, jnp.int32)
            out = []
            for g in range(CH // 16):
                va = plsc.load_gather(zs, [rvecs[g], cols])
                vb = plsc.load_gather(zd, [rvecs[g], cols])
                out.append(carry[g] + va * vb)
            return tuple(out)

        for g in range(CH // 16):
            rbuf[pl.ds(g * 16, 16)] = accs[g]
        pltpu.sync_copy(rbuf, out_hbm.at[pl.ds(off, CH)])


def _dinv_from(d):
    deg = d[0, :, 0:1] + d[1, :, 0:1]
    return lax.rsqrt(deg + 1.0)


def _tc_prescale_mm(xpad, W, deg2):
    def body(x_ref, w_ref, d_ref, o_ref):
        dinv = _dinv_from(d_ref[...])
        xw = jnp.dot(x_ref[...], w_ref[...], preferred_element_type=jnp.float32)
        o_ref[...] = dinv * xw

    return pl.pallas_call(
        body,
        grid=(NP // BR,),
        in_specs=[
            pl.BlockSpec((BR, D), lambda i: (i, 0)),
            pl.BlockSpec((D, H), lambda i: (0, 0)),
            pl.BlockSpec((NC, BR, 16), lambda i: (0, i, 0)),
        ],
        out_specs=pl.BlockSpec((BR, H), lambda i: (i, 0)),
        out_shape=jax.ShapeDtypeStruct((NP, H), jnp.float32),
    )(xpad, W, deg2)


def _tc_layer2(parts1, xws1, deg2, b1r, W2):
    def body(p_ref, x_ref, d_ref, b_ref, w_ref, o_ref):
        dinv = _dinv_from(d_ref[...])
        p = p_ref[...]
        h = jnp.maximum(dinv * (p[0] + p[1] + x_ref[...]) + b_ref[...], 0.0)
        o_ref[...] = dinv * jnp.dot(h, w_ref[...],
                                    preferred_element_type=jnp.float32)

    return pl.pallas_call(
        body,
        grid=(NP // BR,),
        in_specs=[
            pl.BlockSpec((NC, BR, H), lambda i: (0, i, 0)),
            pl.BlockSpec((BR, H), lambda i: (i, 0)),
            pl.BlockSpec((NC, BR, 16), lambda i: (0, i, 0)),
            pl.BlockSpec((1, H), lambda i: (0, 0)),
            pl.BlockSpec((H, H), lambda i: (0, 0)),
        ],
        out_specs=pl.BlockSpec((BR, H), lambda i: (i, 0)),
        out_shape=jax.ShapeDtypeStruct((NP, H), jnp.float32),
    )(parts1, xws1, deg2, b1r, W2)


def _tc_final(parts2, xws2, deg2, b2r):
    def body(p_ref, x_ref, d_ref, b_ref, o_ref):
        dinv = _dinv_from(d_ref[...])
        p = p_ref[...]
        o_ref[...] = dinv * (p[0] + p[1] + x_ref[...]) + b_ref[...]

    return pl.pallas_call(
        body,
        grid=(NP // BR,),
        in_specs=[
            pl.BlockSpec((NC, BR, H), lambda i: (0, i, 0)),
            pl.BlockSpec((BR, H), lambda i: (i, 0)),
            pl.BlockSpec((NC, BR, 16), lambda i: (0, i, 0)),
            pl.BlockSpec((1, H), lambda i: (0, 0)),
        ],
        out_specs=pl.BlockSpec((BR, H), lambda i: (i, 0)),
        out_shape=jax.ShapeDtypeStruct((NP, H), jnp.float32),
    )(parts2, xws2, deg2, b2r)


def kernel(x, edge_index, edge_label_index, W1, b1, W2, b2):
    src = edge_index[0]
    dst = edge_index[1]
    epad = N + (jnp.arange(E_PAD - E, dtype=jnp.int32) % (NP - N))
    src_p = jnp.concatenate([src, epad])
    dst_p = jnp.concatenate([dst, epad])
    ppad = jnp.zeros((P_PAD - P,), jnp.int32)
    sidx = jnp.concatenate([edge_label_index[0], ppad])
    didx = jnp.concatenate([edge_label_index[1], ppad])
    xpad = jnp.pad(x, ((0, NP - N), (0, 0)))
    b1r = b1.reshape(1, H)
    b2r = b2.reshape(1, H)

    deg2 = _sc_degree(dst_p)
    xws1 = _tc_prescale_mm(xpad, W1, deg2)
    parts1 = _sc_edge_pass(src_p, dst_p, xws1)
    xws2 = _tc_layer2(parts1, xws1, deg2, b1r, W2)
    parts2 = _sc_edge_pass(src_p, dst_p, xws2)
    z = _tc_final(parts2, xws2, deg2, b2r)
    res = _sc_decode(sidx, didx, z)
    return res[:P]


# speedup vs baseline: 8.2226x; 1.0140x over previous
import functools

import jax
import jax.numpy as jnp
from jax import lax
from jax.experimental import pallas as pl
from jax.experimental.pallas import tpu as pltpu
from jax.experimental.pallas import tpu_sc as plsc

N = 10000
E = 320000
D = 128
H = 128
P = 100000

NP = 10240
NC = 2
NS = 16
NW = NC * NS
CH = 128
EC = 80
E_PAD = NW * EC * CH
PC = 26
P_PAD = NW * PC * CH
RPT = NP // NS

BR = 1024

_mesh = plsc.VectorSubcoreMesh(core_axis_name="c", subcore_axis_name="s")
_sc_params = pltpu.CompilerParams(needs_layout_passes=False)


@functools.partial(
    pl.kernel,
    out_type=jax.ShapeDtypeStruct((NC, NP, 16), jnp.float32),
    mesh=_mesh,
    compiler_params=_sc_params,
    scratch_types=[
        pltpu.VMEM_SHARED((NP, 16), jnp.float32),
        pltpu.VMEM((2, CH), jnp.int32),
        pltpu.VMEM((CH, 16), jnp.float32),
        pltpu.VMEM((RPT, 16), jnp.float32),
    ],
)
def _sc_degree(dst_hbm, deg_hbm, acc16, idx_v, ones_v, zb):
    cid = lax.axis_index("c")
    sid = lax.axis_index("s")
    wid = sid * NC + cid

    @pl.loop(0, RPT)
    def _(r):
        zb[r] = jnp.zeros((16,), jnp.float32)

    @pl.loop(0, CH)
    def _(r):
        ones_v[r] = jnp.ones((16,), jnp.float32)

    pltpu.sync_copy(zb, acc16.at[pl.ds(sid * RPT, RPT)])
    plsc.subcore_barrier()

    ebase = wid * (EC * CH)

    @pl.loop(0, EC)
    def _(c):
        pltpu.sync_copy(dst_hbm.at[pl.ds(ebase + c * CH, CH)], idx_v.at[0])
        pltpu.sync_copy(ones_v, acc16.at[idx_v.at[0]], add=True)

    plsc.subcore_barrier()
    pltpu.sync_copy(acc16.at[pl.ds(sid * RPT, RPT)],
                    deg_hbm.at[cid, pl.ds(sid * RPT, RPT)])


@functools.partial(
    pl.kernel,
    out_type=jax.ShapeDtypeStruct((NC, NP, H), jnp.float32),
    mesh=_mesh,
    compiler_params=_sc_params,
    scratch_types=[
        pltpu.VMEM_SHARED((NP, H), jnp.float32),
        pltpu.VMEM((2, CH), jnp.int32),
        pltpu.VMEM((2, CH), jnp.int32),
        pltpu.VMEM((CH, H), jnp.float32),
        pltpu.VMEM((CH, H), jnp.float32),
        pltpu.SemaphoreType.DMA,
        pltpu.SemaphoreType.DMA,
        pltpu.SemaphoreType.DMA,
        pltpu.SemaphoreType.DMA,
    ],
)
def _sc_edge_pass(src_hbm, dst_hbm, xws_hbm, parts_hbm,
                  acc, ij_a, ij_b, rows_a, rows_b,
                  gsem_a, gsem_b, isem_a, isem_b):
    cid = lax.axis_index("c")
    sid = lax.axis_index("s")
    wid = sid * NC + cid

    @pl.loop(0, CH)
    def _(r):
        for v in range(H // 16):
            rows_a[r, pl.ds(v * 16, 16)] = jnp.zeros((16,), jnp.float32)

    for k in range(RPT // CH):
        pltpu.sync_copy(rows_a, acc.at[pl.ds(sid * RPT + k * CH, CH)])
    plsc.subcore_barrier()

    rbufs = (rows_a, rows_b)
    gsems = (gsem_a, gsem_b)
    ibufs = (ij_a, ij_b)
    isems = (isem_a, isem_b)
    ebase = wid * EC * CH

    def ij_start(c, buf, sem):
        pltpu.async_copy(src_hbm.at[pl.ds(ebase + c * CH, CH)], buf.at[0], sem)
        pltpu.async_copy(dst_hbm.at[pl.ds(ebase + c * CH, CH)], buf.at[1], sem)

    def ij_wait(c, buf, sem):
        pltpu.make_async_copy(src_hbm.at[pl.ds(ebase + c * CH, CH)],
                              buf.at[0], sem).wait()
        pltpu.make_async_copy(dst_hbm.at[pl.ds(ebase + c * CH, CH)],
                              buf.at[1], sem).wait()

    pltpu.sync_copy(src_hbm.at[pl.ds(ebase, CH)], ij_a.at[0])
    pltpu.sync_copy(dst_hbm.at[pl.ds(ebase, CH)], ij_a.at[1])
    pltpu.sync_copy(src_hbm.at[pl.ds(ebase + CH, CH)], ij_b.at[0])
    pltpu.sync_copy(dst_hbm.at[pl.ds(ebase + CH, CH)], ij_b.at[1])
    pltpu.async_copy(xws_hbm.at[ij_a.at[0]], rows_a, gsem_a)

    @pl.loop(0, EC, step=2)
    def _(c0):
        for b in range(2):
            c = c0 + b

            @pl.when((c > 0) & (c + 1 < EC))
            def _():
                ij_wait(c + 1, ibufs[1 - b], isems[1 - b])

            @pl.when(c + 1 < EC)
            def _():
                pltpu.async_copy(xws_hbm.at[ibufs[1 - b].at[0]],
                                 rbufs[1 - b], gsems[1 - b])

            pltpu.make_async_copy(xws_hbm.at[ibufs[b].at[0]], rbufs[b],
                                  gsems[b]).wait()
            pltpu.sync_copy(rbufs[b], acc.at[ibufs[b].at[1]], add=True)

            @pl.when(c + 2 < EC)
            def _():
                ij_start(c + 2, ibufs[b], isems[b])

    plsc.subcore_barrier()
    pltpu.sync_copy(acc.at[pl.ds(sid * RPT, RPT)],
                    parts_hbm.at[cid, pl.ds(sid * RPT, RPT)])


@functools.partial(
    pl.kernel,
    out_type=jax.ShapeDtypeStruct((P_PAD,), jnp.float32),
    mesh=_mesh,
    compiler_params=_sc_params,
    scratch_types=[
        pltpu.VMEM((PC * CH,), jnp.int32),
        pltpu.VMEM((PC * CH,), jnp.int32),
        pltpu.VMEM((CH, H), jnp.float32),
        pltpu.VMEM((CH, H), jnp.float32),
        pltpu.VMEM((CH, H), jnp.float32),
        pltpu.VMEM((CH, H), jnp.float32),
        pltpu.VMEM((2, CH), jnp.float32),
        pltpu.SemaphoreType.DMA,
        pltpu.SemaphoreType.DMA,
        pltpu.SemaphoreType.DMA,
        pltpu.SemaphoreType.DMA,
    ],
)
def _sc_decode(sidx_hbm, didx_hbm, z_hbm, out_hbm,
               si, di, zs_a, zs_b, zd_a, zd_b, rbuf,
               ssem_a, ssem_b, dsem_a, dsem_b):
    cid = lax.axis_index("c")
    sid = lax.axis_index("s")
    wid = sid * NC + cid
    pbase = wid * PC * CH
    rvecs = [lax.iota(jnp.int32, 16) + 16 * g for g in range(CH // 16)]

    pltpu.sync_copy(sidx_hbm.at[pl.ds(pbase, PC * CH)], si)
    pltpu.sync_copy(didx_hbm.at[pl.ds(pbase, PC * CH)], di)

    def compute(zsb, zdb, b):
        zero = jnp.zeros((16,), jnp.float32)

        @pl.loop(0, H, init_carry=tuple(zero for _ in range(CH // 16)),
                 unroll=4)
        def accs(j, carry):
            cols = jnp.full((16,), j, jnp.int32)
            out = []
            for g in range(CH // 16):
                va = plsc.load_gather(zsb, [rvecs[g], cols])
                vb = plsc.load_gather(zdb, [rvecs[g], cols])
                out.append(carry[g] + va * vb)
            return tuple(out)

        for g in range(CH // 16):
            rbuf[b, pl.ds(g * 16, 16)] = accs[g]

    @pl.loop(0, PC, step=2)
    def _(c0):
        a0 = pltpu.async_copy(z_hbm.at[si.at[pl.ds(c0 * CH, CH)]],
                              zs_a, ssem_a)
        b0 = pltpu.async_copy(z_hbm.at[di.at[pl.ds(c0 * CH, CH)]],
                              zd_a, dsem_a)
        a1 = pltpu.async_copy(z_hbm.at[si.at[pl.ds((c0 + 1) * CH, CH)]],
                              zs_b, ssem_b)
        b1 = pltpu.async_copy(z_hbm.at[di.at[pl.ds((c0 + 1) * CH, CH)]],
                              zd_b, dsem_b)
        a0.wait()
        b0.wait()
        compute(zs_a, zd_a, 0)
        pltpu.sync_copy(rbuf.at[0], out_hbm.at[pl.ds(pbase + c0 * CH, CH)])
        a1.wait()
        b1.wait()
        compute(zs_b, zd_b, 1)
        pltpu.sync_copy(rbuf.at[1],
                        out_hbm.at[pl.ds(pbase + (c0 + 1) * CH, CH)])


def _dinv_from(d):
    deg = d[0, :, 0:1] + d[1, :, 0:1]
    return lax.rsqrt(deg + 1.0)


def _tc_prescale_mm(xpad, W, deg2):
    def body(x_ref, w_ref, d_ref, o_ref):
        dinv = _dinv_from(d_ref[...])
        xw = jnp.dot(x_ref[...], w_ref[...], preferred_element_type=jnp.float32)
        o_ref[...] = dinv * xw

    return pl.pallas_call(
        body,
        grid=(NP // BR,),
        in_specs=[
            pl.BlockSpec((BR, D), lambda i: (i, 0)),
            pl.BlockSpec((D, H), lambda i: (0, 0)),
            pl.BlockSpec((NC, BR, 16), lambda i: (0, i, 0)),
        ],
        out_specs=pl.BlockSpec((BR, H), lambda i: (i, 0)),
        out_shape=jax.ShapeDtypeStruct((NP, H), jnp.float32),
    )(xpad, W, deg2)


def _tc_layer2(parts1, xws1, deg2, b1r, W2):
    def body(p_ref, x_ref, d_ref, b_ref, w_ref, o_ref):
        dinv = _dinv_from(d_ref[...])
        p = p_ref[...]
        h = jnp.maximum(dinv * (p[0] + p[1] + x_ref[...]) + b_ref[...], 0.0)
        o_ref[...] = dinv * jnp.dot(h, w_ref[...],
                                    preferred_element_type=jnp.float32)

    return pl.pallas_call(
        body,
        grid=(NP // BR,),
        in_specs=[
            pl.BlockSpec((NC, BR, H), lambda i: (0, i, 0)),
            pl.BlockSpec((BR, H), lambda i: (i, 0)),
            pl.BlockSpec((NC, BR, 16), lambda i: (0, i, 0)),
            pl.BlockSpec((1, H), lambda i: (0, 0)),
            pl.BlockSpec((H, H), lambda i: (0, 0)),
        ],
        out_specs=pl.BlockSpec((BR, H), lambda i: (i, 0)),
        out_shape=jax.ShapeDtypeStruct((NP, H), jnp.float32),
    )(parts1, xws1, deg2, b1r, W2)


def _tc_final(parts2, xws2, deg2, b2r):
    def body(p_ref, x_ref, d_ref, b_ref, o_ref):
        dinv = _dinv_from(d_ref[...])
        p = p_ref[...]
        o_ref[...] = dinv * (p[0] + p[1] + x_ref[...]) + b_ref[...]

    return pl.pallas_call(
        body,
        grid=(NP // BR,),
        in_specs=[
            pl.BlockSpec((NC, BR, H), lambda i: (0, i, 0)),
            pl.BlockSpec((BR, H), lambda i: (i, 0)),
            pl.BlockSpec((NC, BR, 16), lambda i: (0, i, 0)),
            pl.BlockSpec((1, H), lambda i: (0, 0)),
        ],
        out_specs=pl.BlockSpec((BR, H), lambda i: (i, 0)),
        out_shape=jax.ShapeDtypeStruct((NP, H), jnp.float32),
    )(parts2, xws2, deg2, b2r)


def kernel(x, edge_index, edge_label_index, W1, b1, W2, b2):
    src = edge_index[0]
    dst = edge_index[1]
    epad = N + (jnp.arange(E_PAD - E, dtype=jnp.int32) % (NP - N))
    src_p = jnp.concatenate([src, epad])
    dst_p = jnp.concatenate([dst, epad])
    ppad = jnp.zeros((P_PAD - P,), jnp.int32)
    sidx = jnp.concatenate([edge_label_index[0], ppad])
    didx = jnp.concatenate([edge_label_index[1], ppad])
    xpad = jnp.pad(x, ((0, NP - N), (0, 0)))
    b1r = b1.reshape(1, H)
    b2r = b2.reshape(1, H)

    deg2 = _sc_degree(dst_p)
    xws1 = _tc_prescale_mm(xpad, W1, deg2)
    parts1 = _sc_edge_pass(src_p, dst_p, xws1)
    xws2 = _tc_layer2(parts1, xws1, deg2, b1r, W2)
    parts2 = _sc_edge_pass(src_p, dst_p, xws2)
    z = _tc_final(parts2, xws2, deg2, b2r)
    res = _sc_decode(sidx, didx, z)
    return res[:P]


# speedup vs baseline: 10.7293x; 1.3049x over previous
import functools

import jax
import jax.numpy as jnp
from jax import lax
from jax.experimental import pallas as pl
from jax.experimental.pallas import tpu as pltpu
from jax.experimental.pallas import tpu_sc as plsc

N = 10000
E = 320000
D = 128
H = 128
P = 100000

NP = 10240
NC = 2
NS = 16
NW = NC * NS
CH = 128
EC = 80
E_PAD = NW * EC * CH
PC = 26
P_PAD = NW * PC * CH
RPT = NP // NS

BR = 1024

_mesh = plsc.VectorSubcoreMesh(core_axis_name="c", subcore_axis_name="s")
_sc_params = pltpu.CompilerParams(needs_layout_passes=False)


@functools.partial(
    pl.kernel,
    out_type=jax.ShapeDtypeStruct((NC, NP, 16), jnp.float32),
    mesh=_mesh,
    compiler_params=_sc_params,
    scratch_types=[
        pltpu.VMEM_SHARED((NP, 16), jnp.float32),
        pltpu.VMEM((2, CH), jnp.int32),
        pltpu.VMEM((CH, 16), jnp.float32),
        pltpu.VMEM((RPT, 16), jnp.float32),
    ],
)
def _sc_degree(dst_hbm, deg_hbm, acc16, idx_v, ones_v, zb):
    cid = lax.axis_index("c")
    sid = lax.axis_index("s")
    wid = sid * NC + cid

    @pl.loop(0, RPT)
    def _(r):
        zb[r] = jnp.zeros((16,), jnp.float32)

    @pl.loop(0, CH)
    def _(r):
        ones_v[r] = jnp.ones((16,), jnp.float32)

    pltpu.sync_copy(zb, acc16.at[pl.ds(sid * RPT, RPT)])
    plsc.subcore_barrier()

    ebase = wid * (EC * CH)

    @pl.loop(0, EC)
    def _(c):
        pltpu.sync_copy(dst_hbm.at[pl.ds(ebase + c * CH, CH)], idx_v.at[0])
        pltpu.sync_copy(ones_v, acc16.at[idx_v.at[0]], add=True)

    plsc.subcore_barrier()
    pltpu.sync_copy(acc16.at[pl.ds(sid * RPT, RPT)],
                    deg_hbm.at[cid, pl.ds(sid * RPT, RPT)])


@functools.partial(
    pl.kernel,
    out_type=jax.ShapeDtypeStruct((NC, NP, H), jnp.float32),
    mesh=_mesh,
    compiler_params=_sc_params,
    scratch_types=[
        pltpu.VMEM_SHARED((NP, H), jnp.float32),
        pltpu.VMEM((2, CH), jnp.int32),
        pltpu.VMEM((2, CH), jnp.int32),
        pltpu.VMEM((CH, H), jnp.float32),
        pltpu.VMEM((CH, H), jnp.float32),
        pltpu.SemaphoreType.DMA,
        pltpu.SemaphoreType.DMA,
        pltpu.SemaphoreType.DMA,
        pltpu.SemaphoreType.DMA,
    ],
)
def _sc_edge_pass(src_hbm, dst_hbm, xws_hbm, parts_hbm,
                  acc, ij_a, ij_b, rows_a, rows_b,
                  gsem_a, gsem_b, isem_a, isem_b):
    cid = lax.axis_index("c")
    sid = lax.axis_index("s")
    wid = sid * NC + cid

    @pl.loop(0, CH)
    def _(r):
        for v in range(H // 16):
            rows_a[r, pl.ds(v * 16, 16)] = jnp.zeros((16,), jnp.float32)

    for k in range(RPT // CH):
        pltpu.sync_copy(rows_a, acc.at[pl.ds(sid * RPT + k * CH, CH)])
    plsc.subcore_barrier()

    rbufs = (rows_a, rows_b)
    gsems = (gsem_a, gsem_b)
    ibufs = (ij_a, ij_b)
    isems = (isem_a, isem_b)
    ebase = wid * EC * CH

    def ij_start(c, buf, sem):
        pltpu.async_copy(src_hbm.at[pl.ds(ebase + c * CH, CH)], buf.at[0], sem)
        pltpu.async_copy(dst_hbm.at[pl.ds(ebase + c * CH, CH)], buf.at[1], sem)

    def ij_wait(c, buf, sem):
        pltpu.make_async_copy(src_hbm.at[pl.ds(ebase + c * CH, CH)],
                              buf.at[0], sem).wait()
        pltpu.make_async_copy(dst_hbm.at[pl.ds(ebase + c * CH, CH)],
                              buf.at[1], sem).wait()

    pltpu.sync_copy(src_hbm.at[pl.ds(ebase, CH)], ij_a.at[0])
    pltpu.sync_copy(dst_hbm.at[pl.ds(ebase, CH)], ij_a.at[1])
    pltpu.sync_copy(src_hbm.at[pl.ds(ebase + CH, CH)], ij_b.at[0])
    pltpu.sync_copy(dst_hbm.at[pl.ds(ebase + CH, CH)], ij_b.at[1])
    pltpu.async_copy(xws_hbm.at[ij_a.at[0]], rows_a, gsem_a)

    @pl.loop(0, EC, step=2)
    def _(c0):
        for b in range(2):
            c = c0 + b

            @pl.when((c > 0) & (c + 1 < EC))
            def _():
                ij_wait(c + 1, ibufs[1 - b], isems[1 - b])

            @pl.when(c + 1 < EC)
            def _():
                pltpu.async_copy(xws_hbm.at[ibufs[1 - b].at[0]],
                                 rbufs[1 - b], gsems[1 - b])

            pltpu.make_async_copy(xws_hbm.at[ibufs[b].at[0]], rbufs[b],
                                  gsems[b]).wait()
            pltpu.sync_copy(rbufs[b], acc.at[ibufs[b].at[1]], add=True)

            @pl.when(c + 2 < EC)
            def _():
                ij_start(c + 2, ibufs[b], isems[b])

    plsc.subcore_barrier()
    pltpu.sync_copy(acc.at[pl.ds(sid * RPT, RPT)],
                    parts_hbm.at[cid, pl.ds(sid * RPT, RPT)])


@functools.partial(
    pl.kernel,
    out_type=jax.ShapeDtypeStruct((P_PAD,), jnp.float32),
    mesh=_mesh,
    compiler_params=_sc_params,
    scratch_types=[
        pltpu.VMEM((PC * CH,), jnp.int32),
        pltpu.VMEM((PC * CH,), jnp.int32),
        pltpu.VMEM((CH, H), jnp.float32),
        pltpu.VMEM((CH, H), jnp.float32),
        pltpu.VMEM((CH, H), jnp.float32),
        pltpu.VMEM((CH, H), jnp.float32),
        pltpu.VMEM((2, CH), jnp.float32),
        pltpu.SemaphoreType.DMA,
        pltpu.SemaphoreType.DMA,
        pltpu.SemaphoreType.DMA,
        pltpu.SemaphoreType.DMA,
    ],
)
def _sc_decode(z_hbm, sidx_hbm, didx_hbm, out_hbm,
               si, di, zs_a, zs_b, zd_a, zd_b, rbuf,
               ssem_a, ssem_b, dsem_a, dsem_b):
    cid = lax.axis_index("c")
    sid = lax.axis_index("s")
    wid = sid * NC + cid
    pbase = wid * PC * CH

    pltpu.sync_copy(sidx_hbm.at[pl.ds(pbase, PC * CH)], si)
    pltpu.sync_copy(didx_hbm.at[pl.ds(pbase, PC * CH)], di)

    def _dyn_gather(v, idx):
        dn = lax.GatherDimensionNumbers(
            offset_dims=(), collapsed_slice_dims=(0,), start_index_map=(0,))
        return lax.gather(v, idx[:, None], dn, (1,),
                          mode=lax.GatherScatterMode.PROMISE_IN_BOUNDS)

    shuffles = [jnp.arange(16, dtype=jnp.int32) ^ kk for kk in (8, 4, 2, 1)]
    lmasks = [jnp.arange(16) == l for l in range(16)]

    def compute(zsb, zdb, b):
        @pl.loop(0, CH, step=16)
        def _(p0):
            res = jnp.zeros((16,), jnp.float32)
            for l in range(16):
                p = p0 + l
                acc = zsb[p, pl.ds(0, 16)] * zdb[p, pl.ds(0, 16)]
                for v in range(1, H // 16):
                    acc = acc + (zsb[p, pl.ds(v * 16, 16)]
                                 * zdb[p, pl.ds(v * 16, 16)])
                for sh in shuffles:
                    acc = acc + _dyn_gather(acc, sh)
                res = jnp.where(lmasks[l], acc, res)
            rbuf[b, pl.ds(p0, 16)] = res

    @pl.loop(0, PC, step=2)
    def _(c0):
        a0 = pltpu.async_copy(z_hbm.at[si.at[pl.ds(c0 * CH, CH)]],
                              zs_a, ssem_a)
        b0 = pltpu.async_copy(z_hbm.at[di.at[pl.ds(c0 * CH, CH)]],
                              zd_a, dsem_a)
        a1 = pltpu.async_copy(z_hbm.at[si.at[pl.ds((c0 + 1) * CH, CH)]],
                              zs_b, ssem_b)
        b1 = pltpu.async_copy(z_hbm.at[di.at[pl.ds((c0 + 1) * CH, CH)]],
                              zd_b, dsem_b)
        a0.wait()
        b0.wait()
        compute(zs_a, zd_a, 0)
        pltpu.sync_copy(rbuf.at[0], out_hbm.at[pl.ds(pbase + c0 * CH, CH)])
        a1.wait()
        b1.wait()
        compute(zs_b, zd_b, 1)
        pltpu.sync_copy(rbuf.at[1],
                        out_hbm.at[pl.ds(pbase + (c0 + 1) * CH, CH)])


def _dinv_from(d):
    deg = d[0, :, 0:1] + d[1, :, 0:1]
    return lax.rsqrt(deg + 1.0)


def _tc_prescale_mm(xpad, W, deg2):
    def body(x_ref, w_ref, d_ref, o_ref):
        dinv = _dinv_from(d_ref[...])
        xw = jnp.dot(x_ref[...], w_ref[...], preferred_element_type=jnp.float32)
        o_ref[...] = dinv * xw

    return pl.pallas_call(
        body,
        grid=(NP // BR,),
        in_specs=[
            pl.BlockSpec((BR, D), lambda i: (i, 0)),
            pl.BlockSpec((D, H), lambda i: (0, 0)),
            pl.BlockSpec((NC, BR, 16), lambda i: (0, i, 0)),
        ],
        out_specs=pl.BlockSpec((BR, H), lambda i: (i, 0)),
        out_shape=jax.ShapeDtypeStruct((NP, H), jnp.float32),
    )(xpad, W, deg2)


def _tc_layer2(parts1, xws1, deg2, b1r, W2):
    def body(p_ref, x_ref, d_ref, b_ref, w_ref, o_ref):
        dinv = _dinv_from(d_ref[...])
        p = p_ref[...]
        h = jnp.maximum(dinv * (p[0] + p[1] + x_ref[...]) + b_ref[...], 0.0)
        o_ref[...] = dinv * jnp.dot(h, w_ref[...],
                                    preferred_element_type=jnp.float32)

    return pl.pallas_call(
        body,
        grid=(NP // BR,),
        in_specs=[
            pl.BlockSpec((NC, BR, H), lambda i: (0, i, 0)),
            pl.BlockSpec((BR, H), lambda i: (i, 0)),
            pl.BlockSpec((NC, BR, 16), lambda i: (0, i, 0)),
            pl.BlockSpec((1, H), lambda i: (0, 0)),
            pl.BlockSpec((H, H), lambda i: (0, 0)),
        ],
        out_specs=pl.BlockSpec((BR, H), lambda i: (i, 0)),
        out_shape=jax.ShapeDtypeStruct((NP, H), jnp.float32),
    )(parts1, xws1, deg2, b1r, W2)


def _tc_final(parts2, xws2, deg2, b2r):
    def body(p_ref, x_ref, d_ref, b_ref, o_ref):
        dinv = _dinv_from(d_ref[...])
        p = p_ref[...]
        o_ref[...] = dinv * (p[0] + p[1] + x_ref[...]) + b_ref[...]

    return pl.pallas_call(
        body,
        grid=(NP // BR,),
        in_specs=[
            pl.BlockSpec((NC, BR, H), lambda i: (0, i, 0)),
            pl.BlockSpec((BR, H), lambda i: (i, 0)),
            pl.BlockSpec((NC, BR, 16), lambda i: (0, i, 0)),
            pl.BlockSpec((1, H), lambda i: (0, 0)),
        ],
        out_specs=pl.BlockSpec((BR, H), lambda i: (i, 0)),
        out_shape=jax.ShapeDtypeStruct((NP, H), jnp.float32),
    )(parts2, xws2, deg2, b2r)


def kernel(x, edge_index, edge_label_index, W1, b1, W2, b2):
    src = edge_index[0]
    dst = edge_index[1]
    epad = N + (jnp.arange(E_PAD - E, dtype=jnp.int32) % (NP - N))
    src_p = jnp.concatenate([src, epad])
    dst_p = jnp.concatenate([dst, epad])
    ppad = jnp.zeros((P_PAD - P,), jnp.int32)
    sidx = jnp.concatenate([edge_label_index[0], ppad])
    didx = jnp.concatenate([edge_label_index[1], ppad])
    xpad = jnp.pad(x, ((0, NP - N), (0, 0)))
    b1r = b1.reshape(1, H)
    b2r = b2.reshape(1, H)

    deg2 = _sc_degree(dst_p)
    xws1 = _tc_prescale_mm(xpad, W1, deg2)
    parts1 = _sc_edge_pass(src_p, dst_p, xws1)
    xws2 = _tc_layer2(parts1, xws1, deg2, b1r, W2)
    parts2 = _sc_edge_pass(src_p, dst_p, xws2)
    z = _tc_final(parts2, xws2, deg2, b2r)
    res = _sc_decode(z, sidx, didx)
    return res[:P]
